# Initial kernel scaffold; baseline (speedup 1.0000x reference)
import jax
import jax.numpy as jnp
from jax.experimental import pallas as pl


def kernel(x, edge_index, edge_weight, edge_label_index, neg_edge_index, W1, b1, W2, b2, W3, b3, W4, b4, W5, b5, W6, b6):
    raise NotImplementedError("write your pallas kernel here")


# speedup vs baseline: 1.9039x; 1.9039x over previous
import functools

import jax
import jax.numpy as jnp
from jax import lax
from jax.experimental import pallas as pl
from jax.experimental.pallas import tpu as pltpu
from jax.experimental.pallas import tpu_sc as plsc

_N = 10000
_NPAD = 10240
_RPT = 640
_NC = 2
_NS = 16
_NW = _NC * _NS
_CH = 128

_i32 = jnp.int32
_f32 = jnp.float32


def _iota16():
    return lax.iota(_i32, 16)


def _full16(v):
    return jnp.full((16,), v, _i32)


def _mesh():
    return plsc.VectorSubcoreMesh(core_axis_name="c", subcore_axis_name="s")


def _make_deg_kernel(T):
    @functools.partial(
        pl.kernel,
        out_type=jax.ShapeDtypeStruct((_NC, _NPAD), _f32),
        mesh=_mesh(),
        compiler_params=pltpu.CompilerParams(
            needs_layout_passes=False, use_tc_tiling_on_sc=False),
        scratch_types=[
            pltpu.VMEM((T, _CH), _i32),
            pltpu.VMEM((T, _CH), _f32),
            pltpu.VMEM((_CH,), _f32),
            pltpu.VMEM_SHARED((_NPAD,), _f32),
        ],
    )
    def deg_kernel(dst_hbm, ww_hbm, out_hbm, dst_v, ww_v, zbuf, deg_sp):
        c = lax.axis_index("c")
        s = lax.axis_index("s")
        w = s * _NC + c
        iota = _iota16()
        z16 = jnp.zeros((16,), _f32)
        for j in range(_CH // 16):
            zbuf[pl.ds(j * 16, 16)] = z16
        for k in range(_RPT // _CH):
            pltpu.sync_copy(zbuf, deg_sp.at[pl.ds(s * _RPT + k * _CH, _CH)])
        plsc.subcore_barrier()

        pltpu.sync_copy(dst_hbm.at[pl.ds(w * T, T)], dst_v)
        pltpu.sync_copy(ww_hbm.at[pl.ds(w * T, T)], ww_v)

        def body(t, _):
            pltpu.sync_copy(ww_v.at[t], deg_sp.at[dst_v.at[t]], add=True)
            return 0

        lax.fori_loop(0, T, body, 0)
        plsc.subcore_barrier()
        pltpu.sync_copy(deg_sp.at[pl.ds(s * _RPT, _RPT)],
                        out_hbm.at[c, pl.ds(s * _RPT, _RPT)])

    return deg_kernel


def _make_edge_kernel(T, d):
    ng = _CH // 16

    @functools.partial(
        pl.kernel,
        out_type=jax.ShapeDtypeStruct((_NC, _NPAD, d), _f32),
        mesh=_mesh(),
        compiler_params=pltpu.CompilerParams(
            needs_layout_passes=False, use_tc_tiling_on_sc=False),
        scratch_types=[
            pltpu.VMEM((3, _CH), _i32),
            pltpu.VMEM((3, _CH), _i32),
            pltpu.VMEM((_CH, d), _f32),
            pltpu.VMEM((_CH, d), _f32),
            pltpu.VMEM_SHARED((_NPAD, d), _f32),
            pltpu.SemaphoreType.DMA,
            pltpu.SemaphoreType.DMA,
            pltpu.SemaphoreType.DMA,
            pltpu.SemaphoreType.DMA,
        ],
    )
    def edge_kernel(hd_hbm, em_hbm, out_hbm,
                    m0, m1, rows0, rows1, acc_sp, g0, g1, ms0, ms1):
        c = lax.axis_index("c")
        s = lax.axis_index("s")
        w = s * _NC + c
        iota = _iota16()
        z16 = jnp.zeros((16,), _f32)

        def zrow(i, _):
            for j in range(d // 16):
                rows0[i, pl.ds(j * 16, 16)] = z16
            return 0

        lax.fori_loop(0, _CH, zrow, 0)
        for k in range(_RPT // _CH):
            pltpu.sync_copy(rows0, acc_sp.at[pl.ds(s * _RPT + k * _CH, _CH)])
        plsc.subcore_barrier()

        def start_meta(t, m, sem):
            pltpu.make_async_copy(em_hbm.at[w * T + t], m, sem).start()

        def wait_meta(m, sem):
            pltpu.make_async_copy(em_hbm.at[0], m, sem).wait()

        def start_gather(m, buf, sem):
            pltpu.make_async_copy(hd_hbm.at[m.at[0]], buf, sem).start()

        def wait_gather(m, buf, sem):
            pltpu.make_async_copy(hd_hbm.at[m.at[0]], buf, sem).wait()

        def scale(m, buf):
            wvs = [plsc.bitcast(
                plsc.load_gather(m, [_full16(2), g * 16 + iota]), _f32)
                   for g in range(ng)]

            def col(cix, _):
                cc = _full16(cix)
                for g in range(ng):
                    r = g * 16 + iota
                    v = plsc.load_gather(buf, [r, cc])
                    plsc.store_scatter(buf, [r, cc], v * wvs[g])
                return 0

            lax.fori_loop(0, d, col, 0)

        def scatter_add(m, buf):
            pltpu.sync_copy(buf, acc_sp.at[m.at[1]], add=True)

        start_meta(0, m0, ms0)
        wait_meta(m0, ms0)
        start_gather(m0, rows0, g0)
        start_meta(1, m1, ms1)

        def body(g, _):
            a = 2 * g
            wait_meta(m1, ms1)
            start_gather(m1, rows1, g1)
            wait_gather(m0, rows0, g0)
            scale(m0, rows0)
            scatter_add(m0, rows0)

            @pl.when(g < T // 2 - 1)
            def _():
                start_meta(a + 2, m0, ms0)

            wait_gather(m1, rows1, g1)
            scale(m1, rows1)
            scatter_add(m1, rows1)

            @pl.when(g < T // 2 - 1)
            def _():
                wait_meta(m0, ms0)
                start_gather(m0, rows0, g0)
                start_meta(a + 3, m1, ms1)

            return 0

        lax.fori_loop(0, T // 2, body, 0)
        plsc.subcore_barrier()
        base = s * _RPT
        pltpu.sync_copy(acc_sp.at[pl.ds(base, _RPT)],
                        out_hbm.at[c, pl.ds(base, _RPT)])

    return edge_kernel


def _make_logits_kernel(T, d):
    ng = _CH // 16

    @functools.partial(
        pl.kernel,
        out_type=jax.ShapeDtypeStruct((_NW * T, _CH), _f32),
        mesh=_mesh(),
        compiler_params=pltpu.CompilerParams(
            needs_layout_passes=False, use_tc_tiling_on_sc=False),
        scratch_types=[
            pltpu.VMEM((T, _CH), _i32),
            pltpu.VMEM((_CH,), _i32),
            pltpu.VMEM((_CH,), _i32),
            pltpu.VMEM((_CH,), _i32),
            pltpu.VMEM((_CH,), _i32),
            pltpu.VMEM((_CH, d), _f32),
            pltpu.VMEM((_CH, d), _f32),
            pltpu.VMEM((_CH, d), _f32),
            pltpu.VMEM((_CH, d), _f32),
            pltpu.VMEM((8, _CH), _f32),
            pltpu.VMEM((8, _CH), _f32),
            pltpu.SemaphoreType.DMA,
            pltpu.SemaphoreType.DMA,
            pltpu.SemaphoreType.DMA,
            pltpu.SemaphoreType.DMA,
            pltpu.SemaphoreType.DMA,
            pltpu.SemaphoreType.DMA,
        ],
    )
    def logits_kernel(enc_hbm, pk_hbm, out_hbm,
                      pk_v, ia0, ib0, ia1, ib1, a0, a1, b0, b1, o0, o1,
                      sa0, sa1, sb0, sb1, so0, so1):
        c = lax.axis_index("c")
        s = lax.axis_index("s")
        w = s * _NC + c
        iota = _iota16()
        ias = (ia0, ia1)
        ibs = (ib0, ib1)
        abufs = (a0, a1)
        bbufs = (b0, b1)
        obufs = (o0, o1)
        sos = (so0, so1)

        pltpu.sync_copy(pk_hbm.at[pl.ds(w * T, T)], pk_v)

        def unpack(t, ia, ib):
            tt = _full16(t)
            for j in range(ng):
                pk = plsc.load_gather(pk_v, [tt, j * 16 + iota])
                ia[pl.ds(j * 16, 16)] = jnp.bitwise_and(pk, 0xFFFF)
                ib[pl.ds(j * 16, 16)] = lax.shift_right_logical(pk, 16)

        def start_gathers(p):
            pltpu.make_async_copy(enc_hbm.at[ias[p]], abufs[p], sa0 if p == 0 else sa1).start()
            pltpu.make_async_copy(enc_hbm.at[ibs[p]], bbufs[p], sb0 if p == 0 else sb1).start()

        def wait_gathers(p):
            pltpu.make_async_copy(enc_hbm.at[ias[p]], abufs[p], sa0 if p == 0 else sa1).wait()
            pltpu.make_async_copy(enc_hbm.at[ibs[p]], bbufs[p], sb0 if p == 0 else sb1).wait()

        def process(p, orow, obuf):
            abuf, bbuf = abufs[p], bbufs[p]

            def col(cix, accs):
                cc = _full16(cix)
                out = []
                for g in range(ng):
                    r = g * 16 + iota
                    va = plsc.load_gather(abuf, [r, cc])
                    vb = plsc.load_gather(bbuf, [r, cc])
                    out.append(accs[g] + va * vb)
                return tuple(out)

            accs = lax.fori_loop(0, d, col,
                                 tuple(jnp.zeros((16,), _f32) for _ in range(ng)))
            for g in range(ng):
                obuf[orow, pl.ds(g * 16, 16)] = 1.0 / (1.0 + jnp.exp(-accs[g]))

        def flush_out(q, m, half):
            pltpu.make_async_copy(
                obufs[q], out_hbm.at[pl.ds(w * T + 16 * m + 8 * half, 8)],
                sos[q]).start()

        def wait_out(q):
            pltpu.make_async_copy(obufs[q], out_hbm.at[pl.ds(0, 8)],
                                  sos[q]).wait()

        unpack(0, ia0, ib0)
        start_gathers(0)

        nm = T // 16

        def body(m, _):
            t0 = 16 * m
            for j in range(16):
                p = j % 2
                q = (j // 8) % 2
                t = t0 + j
                wait_gathers(p)

                @pl.when(jnp.logical_or(m < nm - 1, j < 15))
                def _():
                    unpack(t + 1, ias[1 - p], ibs[1 - p])
                    start_gathers(1 - p)

                if j % 8 == 0:
                    @pl.when(m > 0)
                    def _():
                        wait_out(q)

                process(p, j % 8, obufs[q])
                if j % 8 == 7:
                    flush_out(q, m, j // 8)
            return 0

        lax.fori_loop(0, nm, body, 0)
        wait_out(0)
        wait_out(1)

    return logits_kernel


_BR = 640


def _tc_first(pd_t, xp, w1):
    def body(pd_ref, x_ref, w_ref, dis_ref, hd_ref):
        deg = pd_ref[:, 0:1] + pd_ref[:, 1:2] + 1.0
        dis = jnp.where(deg > 0,
                        lax.rsqrt(jnp.maximum(deg, 1e-12)), 0.0)
        dis_ref[...] = dis
        hd_ref[...] = dis * jnp.dot(x_ref[...], w_ref[...],
                                    preferred_element_type=_f32)

    return pl.pallas_call(
        body,
        grid=(_NPAD // _BR,),
        in_specs=[
            pl.BlockSpec((_BR, 2), lambda i: (i, 0)),
            pl.BlockSpec((_BR, 128), lambda i: (i, 0)),
            pl.BlockSpec((128, 128), lambda i: (0, 0)),
        ],
        out_specs=[
            pl.BlockSpec((_BR, 1), lambda i: (i, 0)),
            pl.BlockSpec((_BR, 128), lambda i: (i, 0)),
        ],
        out_shape=[
            jax.ShapeDtypeStruct((_NPAD, 1), _f32),
            jax.ShapeDtypeStruct((_NPAD, 128), _f32),
        ],
    )(pd_t, xp, w1)


def _tc_combine_mm(pa, pb, hd, dis, b2d, wn):
    dh = pa.shape[2]
    d = hd.shape[1]
    dn = wn.shape[1]

    def body(pa_ref, pb_ref, hd_ref, dis_ref, b_ref, w_ref, out_ref):
        sacc = hd_ref[...] + jnp.concatenate(
            [pa_ref[0] + pa_ref[1], pb_ref[0] + pb_ref[1]], axis=-1)
        xr = jnp.maximum(dis_ref[...] * sacc + b_ref[...], 0.0)
        out_ref[...] = dis_ref[...] * jnp.dot(xr, w_ref[...],
                                              preferred_element_type=_f32)

    return pl.pallas_call(
        body,
        grid=(_NPAD // _BR,),
        in_specs=[
            pl.BlockSpec((2, _BR, dh), lambda i: (0, i, 0)),
            pl.BlockSpec((2, _BR, dh), lambda i: (0, i, 0)),
            pl.BlockSpec((_BR, d), lambda i: (i, 0)),
            pl.BlockSpec((_BR, 1), lambda i: (i, 0)),
            pl.BlockSpec((1, d), lambda i: (0, 0)),
            pl.BlockSpec((d, dn), lambda i: (0, 0)),
        ],
        out_specs=pl.BlockSpec((_BR, dn), lambda i: (i, 0)),
        out_shape=jax.ShapeDtypeStruct((_NPAD, dn), _f32),
    )(pa, pb, hd, dis, b2d, wn)


def _tc_combine_last(p, hd, dis, b2d):
    d = hd.shape[1]

    def body(p_ref, hd_ref, dis_ref, b_ref, out_ref):
        sacc = p_ref[0] + p_ref[1] + hd_ref[...]
        out_ref[...] = dis_ref[...] * sacc + b_ref[...]

    return pl.pallas_call(
        body,
        grid=(_NPAD // _BR,),
        in_specs=[
            pl.BlockSpec((2, _BR, d), lambda i: (0, i, 0)),
            pl.BlockSpec((_BR, d), lambda i: (i, 0)),
            pl.BlockSpec((_BR, 1), lambda i: (i, 0)),
            pl.BlockSpec((1, d), lambda i: (0, 0)),
        ],
        out_specs=pl.BlockSpec((_BR, d), lambda i: (i, 0)),
        out_shape=jax.ShapeDtypeStruct((_NPAD, d), _f32),
    )(p, hd, dis, b2d)


def _pad_chunks(a, t, fill):
    n = _NW * t * _CH
    pad = n - a.shape[0]
    return jnp.concatenate(
        [a, jnp.full((pad,), fill, a.dtype)]).reshape(_NW * t, _CH)


def kernel(x, edge_index, edge_weight, edge_label_index, neg_edge_index,
           W1, b1, W2, b2, W3, b3, W4, b4, W5, b5, W6, b6):
    e = edge_index.shape[1]
    te = -(-(-(-e // _NW)) // _CH)
    te = te + (te % 2)
    l2 = edge_label_index.shape[1] + neg_edge_index.shape[1]
    tl = -(-(-(-l2 // _NW)) // _CH)
    tl = ((tl + 15) // 16) * 16

    src2d = _pad_chunks(edge_index[0], te, 0)
    dst2d = _pad_chunks(edge_index[1], te, 0)
    ww2d = _pad_chunks(edge_weight, te, 0.0)
    em = jnp.stack(
        [src2d, dst2d, lax.bitcast_convert_type(ww2d, _i32)], axis=1)

    eli = jnp.concatenate([edge_label_index, neg_edge_index], axis=1)
    pk2d = _pad_chunks(eli[0] + eli[1] * 65536, tl, 0)

    xp = jnp.concatenate(
        [x, jnp.zeros((_NPAD - x.shape[0], x.shape[1]), _f32)])

    deg_k = _make_deg_kernel(te)
    edge_k = _make_edge_kernel(te, 64)
    logits_k = _make_logits_kernel(tl, 64)

    pd = deg_k(dst2d, ww2d)
    dis, hd = _tc_first(pd.T, xp, W1)

    ws = [W2, W3, W4, W5, W6]
    bs = [b1, b2, b3, b4, b5]
    for i in range(5):
        pa = edge_k(hd[:, :64], em)
        pb = edge_k(hd[:, 64:], em)
        hd = _tc_combine_mm(pa, pb, hd, dis, bs[i].reshape(1, -1), ws[i])

    p6 = edge_k(hd, em)
    enc = _tc_combine_last(p6, hd, dis, b6.reshape(1, -1))

    logits2d = logits_k(enc, pk2d)
    return logits2d.reshape(-1)[:l2]


# speedup vs baseline: 2.0601x; 1.0821x over previous
import functools

import jax
import jax.numpy as jnp
from jax import lax
from jax.experimental import pallas as pl
from jax.experimental.pallas import tpu as pltpu
from jax.experimental.pallas import tpu_sc as plsc

_N = 10000
_NPAD = 10240
_RPT = 640
_NC = 2
_NS = 16
_NW = _NC * _NS
_CH = 128

_i32 = jnp.int32
_f32 = jnp.float32


def _iota16():
    return lax.iota(_i32, 16)


def _full16(v):
    return jnp.full((16,), v, _i32)


def _mesh():
    return plsc.VectorSubcoreMesh(core_axis_name="c", subcore_axis_name="s")


def _make_deg_kernel(T):
    @functools.partial(
        pl.kernel,
        out_type=jax.ShapeDtypeStruct((_NC, _NPAD), _f32),
        mesh=_mesh(),
        compiler_params=pltpu.CompilerParams(
            needs_layout_passes=False, use_tc_tiling_on_sc=False),
        scratch_types=[
            pltpu.VMEM((T, _CH), _i32),
            pltpu.VMEM((T, _CH), _f32),
            pltpu.VMEM((_CH,), _f32),
            pltpu.VMEM_SHARED((_NPAD,), _f32),
        ],
    )
    def deg_kernel(dst_hbm, ww_hbm, out_hbm, dst_v, ww_v, zbuf, deg_sp):
        c = lax.axis_index("c")
        s = lax.axis_index("s")
        w = s * _NC + c
        iota = _iota16()
        z16 = jnp.zeros((16,), _f32)
        for j in range(_CH // 16):
            zbuf[pl.ds(j * 16, 16)] = z16
        for k in range(_RPT // _CH):
            pltpu.sync_copy(zbuf, deg_sp.at[pl.ds(s * _RPT + k * _CH, _CH)])
        plsc.subcore_barrier()

        pltpu.sync_copy(dst_hbm.at[pl.ds(w * T, T)], dst_v)
        pltpu.sync_copy(ww_hbm.at[pl.ds(w * T, T)], ww_v)

        def body(t, _):
            pltpu.sync_copy(ww_v.at[t], deg_sp.at[dst_v.at[t]], add=True)
            return 0

        lax.fori_loop(0, T, body, 0)
        plsc.subcore_barrier()
        pltpu.sync_copy(deg_sp.at[pl.ds(s * _RPT, _RPT)],
                        out_hbm.at[c, pl.ds(s * _RPT, _RPT)])

    return deg_kernel


def _make_edge_kernel(T, d):
    ng = _CH // 16

    nr = 4
    nq = 8

    @functools.partial(
        pl.kernel,
        out_type=jax.ShapeDtypeStruct((_NC, _NPAD, d), _f32),
        mesh=_mesh(),
        compiler_params=pltpu.CompilerParams(
            needs_layout_passes=False, use_tc_tiling_on_sc=False,
            internal_scratch_in_bytes=1024),
        scratch_types=(
            [pltpu.VMEM((3, _CH), _i32)] * nq
            + [pltpu.VMEM((_CH, d), _f32)] * nr
            + [pltpu.VMEM_SHARED((_NPAD, d), _f32)]
            + [pltpu.SemaphoreType.DMA] * (nq + 2 * nr)
        ),
    )
    def edge_kernel(hd_hbm, em_hbm, out_hbm, *scr):
        ms = scr[:nq]
        rows = scr[nq:nq + nr]
        acc_sp = scr[nq + nr]
        msem = scr[nq + nr + 1:nq + nr + 1 + nq]
        gsem = scr[nq + nr + 1 + nq:nq + nr + 1 + nq + nr]
        ssem = scr[nq + nr + 1 + nq + nr:]
        c = lax.axis_index("c")
        s = lax.axis_index("s")
        w = s * _NC + c
        iota = _iota16()
        z16 = jnp.zeros((16,), _f32)

        def zrow(i, _):
            for j in range(d // 16):
                rows[0][i, pl.ds(j * 16, 16)] = z16
            return 0

        lax.fori_loop(0, _CH, zrow, 0)
        for k in range(_RPT // _CH):
            pltpu.sync_copy(rows[0], acc_sp.at[pl.ds(s * _RPT + k * _CH, _CH)])
        plsc.subcore_barrier()

        def start_meta(t, u):
            pltpu.make_async_copy(em_hbm.at[w * T + t], ms[u], msem[u]).start()

        def wait_meta(u):
            pltpu.make_async_copy(em_hbm.at[0], ms[u], msem[u]).wait()

        def start_gather(u, k):
            pltpu.make_async_copy(hd_hbm.at[ms[u].at[0]], rows[k],
                                  gsem[k]).start()

        def wait_gather(u, k):
            pltpu.make_async_copy(hd_hbm.at[ms[u].at[0]], rows[k],
                                  gsem[k]).wait()

        def start_scatter(u, k):
            pltpu.async_copy(rows[k], acc_sp.at[ms[u].at[1]],
                             ssem[k], add=True)

        def wait_scatter(u, k):
            pltpu.make_async_copy(rows[k], acc_sp.at[ms[u].at[1]],
                                  ssem[k]).wait()

        def scale(u, k):
            m, buf = ms[u], rows[k]
            wvs = [plsc.bitcast(
                plsc.load_gather(m, [_full16(2), g * 16 + iota]), _f32)
                   for g in range(ng)]

            def col(cix, _):
                cc = _full16(cix)
                for g in range(ng):
                    r = g * 16 + iota
                    v = plsc.load_gather(buf, [r, cc])
                    plsc.store_scatter(buf, [r, cc], v * wvs[g])
                return 0

            lax.fori_loop(0, d, col, 0, unroll=4)

        for t in range(nq - 2):
            start_meta(t, t)
        wait_meta(0)
        start_gather(0, 0)
        wait_meta(1)
        start_gather(1, 1)

        def body(b, _):
            t0 = b * nq
            for j in range(nq):
                u = j
                k = j % nr
                tt = t0 + j
                wait_gather(u, k)
                scale(u, k)
                start_scatter(u, k)

                @pl.when(tt + 2 < T)
                def _(tt=tt):
                    u2 = (j + 2) % nq
                    k2 = (j + 2) % nr
                    if j < 2:
                        @pl.when(b > 0)
                        def _():
                            wait_scatter((j + 6) % nq, k2)
                    else:
                        wait_scatter((j - 2) % nq, k2)
                    wait_meta(u2)
                    start_gather(u2, k2)

                @pl.when(tt + nq - 2 < T)
                def _(tt=tt, u=u):
                    start_meta(tt + nq - 2, (j + nq - 2) % nq)
            return 0

        lax.fori_loop(0, T // nq, body, 0)
        for t in range(T - 4, T):
            wait_scatter(t % nq, t % nr)
        plsc.subcore_barrier()
        base = s * _RPT
        pltpu.sync_copy(acc_sp.at[pl.ds(base, _RPT)],
                        out_hbm.at[c, pl.ds(base, _RPT)])

    return edge_kernel


def _make_logits_kernel(T, d):
    ng = _CH // 16
    nr = 4
    no = 3
    nb = 24

    @functools.partial(
        pl.kernel,
        out_type=jax.ShapeDtypeStruct((_NW * T, _CH), _f32),
        mesh=_mesh(),
        compiler_params=pltpu.CompilerParams(
            needs_layout_passes=False, use_tc_tiling_on_sc=False,
            internal_scratch_in_bytes=1024),
        scratch_types=(
            [pltpu.VMEM((T, _CH), _i32)]
            + [pltpu.VMEM((_CH,), _i32)] * (2 * nr)
            + [pltpu.VMEM((_CH, d), _f32)] * (2 * nr)
            + [pltpu.VMEM((8, _CH), _f32)] * no
            + [pltpu.SemaphoreType.DMA] * (2 * nr + no)
        ),
    )
    def logits_kernel(enc_hbm, pk_hbm, out_hbm, *scr):
        pk_v = scr[0]
        ias = scr[1:1 + nr]
        ibs = scr[1 + nr:1 + 2 * nr]
        abufs = scr[1 + 2 * nr:1 + 3 * nr]
        bbufs = scr[1 + 3 * nr:1 + 4 * nr]
        obufs = scr[1 + 4 * nr:1 + 4 * nr + no]
        sa = scr[1 + 4 * nr + no:1 + 5 * nr + no]
        sb = scr[1 + 5 * nr + no:1 + 6 * nr + no]
        so = scr[1 + 6 * nr + no:]
        c = lax.axis_index("c")
        s = lax.axis_index("s")
        w = s * _NC + c
        iota = _iota16()

        pltpu.sync_copy(pk_hbm.at[pl.ds(w * T, T)], pk_v)

        def unpack(t, k):
            tt = _full16(t)
            ia, ib = ias[k], ibs[k]
            for j in range(ng):
                pk = plsc.load_gather(pk_v, [tt, j * 16 + iota])
                ia[pl.ds(j * 16, 16)] = jnp.bitwise_and(pk, 0xFFFF)
                ib[pl.ds(j * 16, 16)] = lax.shift_right_logical(pk, 16)

        def start_gathers(k):
            pltpu.make_async_copy(enc_hbm.at[ias[k]], abufs[k], sa[k]).start()
            pltpu.make_async_copy(enc_hbm.at[ibs[k]], bbufs[k], sb[k]).start()

        def wait_gathers(k):
            pltpu.make_async_copy(enc_hbm.at[ias[k]], abufs[k], sa[k]).wait()
            pltpu.make_async_copy(enc_hbm.at[ibs[k]], bbufs[k], sb[k]).wait()

        def process(k, q, orow):
            abuf, bbuf, obuf = abufs[k], bbufs[k], obufs[q]

            def col(cix, accs):
                cc = _full16(cix)
                out = []
                for g in range(ng):
                    r = g * 16 + iota
                    va = plsc.load_gather(abuf, [r, cc])
                    vb = plsc.load_gather(bbuf, [r, cc])
                    out.append(accs[g] + va * vb)
                return tuple(out)

            accs = lax.fori_loop(0, d, col,
                                 tuple(jnp.zeros((16,), _f32)
                                       for _ in range(ng)), unroll=4)
            for g in range(ng):
                obuf[orow, pl.ds(g * 16, 16)] = 1.0 / (1.0 + jnp.exp(-accs[g]))

        def flush_out(q, t):
            pltpu.make_async_copy(
                obufs[q], out_hbm.at[pl.ds(w * T + t - 7, 8)], so[q]).start()

        def wait_out(q):
            pltpu.make_async_copy(obufs[q], out_hbm.at[pl.ds(0, 8)],
                                  so[q]).wait()

        unpack(0, 0)
        start_gathers(0)
        unpack(1, 1)
        start_gathers(1)

        def body(b, _):
            t0 = b * nb
            for j in range(nb):
                k = j % nr
                q = (j // 8) % no
                tt = t0 + j
                wait_gathers(k)

                @pl.when(tt + 2 < T)
                def _(tt=tt):
                    k2 = (j + 2) % nr
                    unpack(tt + 2, k2)
                    start_gathers(k2)

                if j % 8 == 0:
                    @pl.when(b > 0)
                    def _():
                        wait_out(q)

                process(k, q, j % 8)
                if j % 8 == 7:
                    flush_out(q, tt)
            return 0

        lax.fori_loop(0, T // nb, body, 0)
        for q in range(no):
            wait_out(q)

    return logits_kernel


_BR = 640


def _tc_first(pd_t, xp, w1):
    def body(pd_ref, x_ref, w_ref, dis_ref, hd_ref):
        deg = pd_ref[:, 0:1] + pd_ref[:, 1:2] + 1.0
        dis = jnp.where(deg > 0,
                        lax.rsqrt(jnp.maximum(deg, 1e-12)), 0.0)
        dis_ref[...] = dis
        hd_ref[...] = dis * jnp.dot(x_ref[...], w_ref[...],
                                    preferred_element_type=_f32)

    return pl.pallas_call(
        body,
        grid=(_NPAD // _BR,),
        in_specs=[
            pl.BlockSpec((_BR, 2), lambda i: (i, 0)),
            pl.BlockSpec((_BR, 128), lambda i: (i, 0)),
            pl.BlockSpec((128, 128), lambda i: (0, 0)),
        ],
        out_specs=[
            pl.BlockSpec((_BR, 1), lambda i: (i, 0)),
            pl.BlockSpec((_BR, 128), lambda i: (i, 0)),
        ],
        out_shape=[
            jax.ShapeDtypeStruct((_NPAD, 1), _f32),
            jax.ShapeDtypeStruct((_NPAD, 128), _f32),
        ],
    )(pd_t, xp, w1)


def _tc_combine_mm(pa, pb, hd, dis, b2d, wn):
    dh = pa.shape[2]
    d = hd.shape[1]
    dn = wn.shape[1]

    def body(pa_ref, pb_ref, hd_ref, dis_ref, b_ref, w_ref, out_ref):
        sacc = hd_ref[...] + jnp.concatenate(
            [pa_ref[0] + pa_ref[1], pb_ref[0] + pb_ref[1]], axis=-1)
        xr = jnp.maximum(dis_ref[...] * sacc + b_ref[...], 0.0)
        out_ref[...] = dis_ref[...] * jnp.dot(xr, w_ref[...],
                                              preferred_element_type=_f32)

    return pl.pallas_call(
        body,
        grid=(_NPAD // _BR,),
        in_specs=[
            pl.BlockSpec((2, _BR, dh), lambda i: (0, i, 0)),
            pl.BlockSpec((2, _BR, dh), lambda i: (0, i, 0)),
            pl.BlockSpec((_BR, d), lambda i: (i, 0)),
            pl.BlockSpec((_BR, 1), lambda i: (i, 0)),
            pl.BlockSpec((1, d), lambda i: (0, 0)),
            pl.BlockSpec((d, dn), lambda i: (0, 0)),
        ],
        out_specs=pl.BlockSpec((_BR, dn), lambda i: (i, 0)),
        out_shape=jax.ShapeDtypeStruct((_NPAD, dn), _f32),
    )(pa, pb, hd, dis, b2d, wn)


def _tc_combine_last(p, hd, dis, b2d):
    d = hd.shape[1]

    def body(p_ref, hd_ref, dis_ref, b_ref, out_ref):
        sacc = p_ref[0] + p_ref[1] + hd_ref[...]
        out_ref[...] = dis_ref[...] * sacc + b_ref[...]

    return pl.pallas_call(
        body,
        grid=(_NPAD // _BR,),
        in_specs=[
            pl.BlockSpec((2, _BR, d), lambda i: (0, i, 0)),
            pl.BlockSpec((_BR, d), lambda i: (i, 0)),
            pl.BlockSpec((_BR, 1), lambda i: (i, 0)),
            pl.BlockSpec((1, d), lambda i: (0, 0)),
        ],
        out_specs=pl.BlockSpec((_BR, d), lambda i: (i, 0)),
        out_shape=jax.ShapeDtypeStruct((_NPAD, d), _f32),
    )(p, hd, dis, b2d)


def _pad_chunks(a, t, fill):
    n = _NW * t * _CH
    pad = n - a.shape[0]
    return jnp.concatenate(
        [a, jnp.full((pad,), fill, a.dtype)]).reshape(_NW * t, _CH)


def kernel(x, edge_index, edge_weight, edge_label_index, neg_edge_index,
           W1, b1, W2, b2, W3, b3, W4, b4, W5, b5, W6, b6):
    e = edge_index.shape[1]
    te = -(-(-(-e // _NW)) // _CH)
    te = ((te + 7) // 8) * 8
    l2 = edge_label_index.shape[1] + neg_edge_index.shape[1]
    tl = -(-(-(-l2 // _NW)) // _CH)
    tl = ((tl + 23) // 24) * 24

    src2d = _pad_chunks(edge_index[0], te, 0)
    dst2d = _pad_chunks(edge_index[1], te, 0)
    ww2d = _pad_chunks(edge_weight, te, 0.0)
    em = jnp.stack(
        [src2d, dst2d, lax.bitcast_convert_type(ww2d, _i32)], axis=1)

    eli = jnp.concatenate([edge_label_index, neg_edge_index], axis=1)
    pk2d = _pad_chunks(eli[0] + eli[1] * 65536, tl, 0)

    xp = jnp.concatenate(
        [x, jnp.zeros((_NPAD - x.shape[0], x.shape[1]), _f32)])

    deg_k = _make_deg_kernel(te)
    edge_k = _make_edge_kernel(te, 64)
    logits_k = _make_logits_kernel(tl, 64)

    pd = deg_k(dst2d, ww2d)
    dis, hd = _tc_first(pd.T, xp, W1)

    ws = [W2, W3, W4, W5, W6]
    bs = [b1, b2, b3, b4, b5]
    for i in range(5):
        pa = edge_k(hd[:, :64], em)
        pb = edge_k(hd[:, 64:], em)
        hd = _tc_combine_mm(pa, pb, hd, dis, bs[i].reshape(1, -1), ws[i])

    p6 = edge_k(hd, em)
    enc = _tc_combine_last(p6, hd, dis, b6.reshape(1, -1))

    logits2d = logits_k(enc, pk2d)
    return logits2d.reshape(-1)[:l2]


# speedup vs baseline: 3.8824x; 1.8846x over previous
import functools

import jax
import jax.numpy as jnp
from jax import lax
from jax.experimental import pallas as pl
from jax.experimental.pallas import tpu as pltpu
from jax.experimental.pallas import tpu_sc as plsc

_N = 10000
_NPAD = 10240
_RPT = 640
_NC = 2
_NS = 16
_NW = _NC * _NS
_CH = 128

_i32 = jnp.int32
_f32 = jnp.float32


def _iota16():
    return lax.iota(_i32, 16)


def _full16(v):
    return jnp.full((16,), v, _i32)


def _mesh():
    return plsc.VectorSubcoreMesh(core_axis_name="c", subcore_axis_name="s")


def _make_deg_kernel(T):
    @functools.partial(
        pl.kernel,
        out_type=jax.ShapeDtypeStruct((_NC, _NPAD), _f32),
        mesh=_mesh(),
        compiler_params=pltpu.CompilerParams(
            needs_layout_passes=False, use_tc_tiling_on_sc=False),
        scratch_types=[
            pltpu.VMEM((T, _CH), _i32),
            pltpu.VMEM((T, _CH), _f32),
            pltpu.VMEM((_CH,), _f32),
            pltpu.VMEM_SHARED((_NPAD,), _f32),
        ],
    )
    def deg_kernel(dst_hbm, ww_hbm, out_hbm, dst_v, ww_v, zbuf, deg_sp):
        c = lax.axis_index("c")
        s = lax.axis_index("s")
        w = s * _NC + c
        iota = _iota16()
        z16 = jnp.zeros((16,), _f32)
        for j in range(_CH // 16):
            zbuf[pl.ds(j * 16, 16)] = z16
        for k in range(_RPT // _CH):
            pltpu.sync_copy(zbuf, deg_sp.at[pl.ds(s * _RPT + k * _CH, _CH)])
        plsc.subcore_barrier()

        pltpu.sync_copy(dst_hbm.at[pl.ds(w * T, T)], dst_v)
        pltpu.sync_copy(ww_hbm.at[pl.ds(w * T, T)], ww_v)

        def body(t, _):
            pltpu.sync_copy(ww_v.at[t], deg_sp.at[dst_v.at[t]], add=True)
            return 0

        lax.fori_loop(0, T, body, 0)
        plsc.subcore_barrier()
        pltpu.sync_copy(deg_sp.at[pl.ds(s * _RPT, _RPT)],
                        out_hbm.at[c, pl.ds(s * _RPT, _RPT)])

    return deg_kernel


def _make_edge_kernel(T, d):
    ng = _CH // 16

    nr = 4
    nq = 8

    @functools.partial(
        pl.kernel,
        out_type=jax.ShapeDtypeStruct((_NC, _NPAD, d), _f32),
        mesh=_mesh(),
        compiler_params=pltpu.CompilerParams(
            needs_layout_passes=False, use_tc_tiling_on_sc=False,
            internal_scratch_in_bytes=1024),
        scratch_types=(
            [pltpu.VMEM((3, _CH), _i32)] * nq
            + [pltpu.VMEM((_CH, d), _f32)] * nr
            + [pltpu.VMEM_SHARED((_NPAD, d), _f32)]
            + [pltpu.SemaphoreType.DMA] * (nq + 2 * nr)
        ),
    )
    def edge_kernel(hd_hbm, em_hbm, out_hbm, *scr):
        ms = scr[:nq]
        rows = scr[nq:nq + nr]
        acc_sp = scr[nq + nr]
        msem = scr[nq + nr + 1:nq + nr + 1 + nq]
        gsem = scr[nq + nr + 1 + nq:nq + nr + 1 + nq + nr]
        ssem = scr[nq + nr + 1 + nq + nr:]
        c = lax.axis_index("c")
        s = lax.axis_index("s")
        w = s * _NC + c
        iota = _iota16()
        z16 = jnp.zeros((16,), _f32)

        def zrow(i, _):
            for j in range(d // 16):
                rows[0][i, pl.ds(j * 16, 16)] = z16
            return 0

        lax.fori_loop(0, _CH, zrow, 0)
        for k in range(_RPT // _CH):
            pltpu.sync_copy(rows[0], acc_sp.at[pl.ds(s * _RPT + k * _CH, _CH)])
        plsc.subcore_barrier()

        def start_meta(t, u):
            pltpu.make_async_copy(em_hbm.at[w * T + t], ms[u], msem[u]).start()

        def wait_meta(u):
            pltpu.make_async_copy(em_hbm.at[0], ms[u], msem[u]).wait()

        def start_gather(u, k):
            pltpu.make_async_copy(hd_hbm.at[ms[u].at[0]], rows[k],
                                  gsem[k]).start()

        def wait_gather(u, k):
            pltpu.make_async_copy(hd_hbm.at[ms[u].at[0]], rows[k],
                                  gsem[k]).wait()

        def start_scatter(u, k):
            pltpu.async_copy(rows[k], acc_sp.at[ms[u].at[1]],
                             ssem[k], add=True)

        def wait_scatter(u, k):
            pltpu.make_async_copy(rows[k], acc_sp.at[ms[u].at[1]],
                                  ssem[k]).wait()

        def scale(u, k):
            m, buf = ms[u], rows[k]
            wvs = [plsc.bitcast(
                plsc.load_gather(m, [_full16(2), g * 16 + iota]), _f32)
                   for g in range(ng)]

            @plsc.parallel_loop(0, d, unroll=4)
            def _(cix):
                cc = _full16(cix)
                for g in range(ng):
                    r = g * 16 + iota
                    v = plsc.load_gather(buf, [r, cc])
                    plsc.store_scatter(buf, [r, cc], v * wvs[g])

        for t in range(nq - 2):
            start_meta(t, t)
        wait_meta(0)
        start_gather(0, 0)
        wait_meta(1)
        start_gather(1, 1)

        def body(b, _):
            t0 = b * nq
            for j in range(nq):
                u = j
                k = j % nr
                tt = t0 + j
                wait_gather(u, k)
                scale(u, k)
                start_scatter(u, k)

                @pl.when(tt + 2 < T)
                def _(tt=tt):
                    u2 = (j + 2) % nq
                    k2 = (j + 2) % nr
                    if j < 2:
                        @pl.when(b > 0)
                        def _():
                            wait_scatter((j + 6) % nq, k2)
                    else:
                        wait_scatter((j - 2) % nq, k2)
                    wait_meta(u2)
                    start_gather(u2, k2)

                @pl.when(tt + nq - 2 < T)
                def _(tt=tt, u=u):
                    start_meta(tt + nq - 2, (j + nq - 2) % nq)
            return 0

        lax.fori_loop(0, T // nq, body, 0)
        for t in range(T - 4, T):
            wait_scatter(t % nq, t % nr)
        plsc.subcore_barrier()
        base = s * _RPT
        pltpu.sync_copy(acc_sp.at[pl.ds(base, _RPT)],
                        out_hbm.at[c, pl.ds(base, _RPT)])

    return edge_kernel


def _make_logits_kernel(T, d):
    ng = _CH // 16
    nr = 6
    no = 3
    nb = 24

    @functools.partial(
        pl.kernel,
        out_type=jax.ShapeDtypeStruct((_NW * T, _CH), _f32),
        mesh=_mesh(),
        compiler_params=pltpu.CompilerParams(
            needs_layout_passes=False, use_tc_tiling_on_sc=False,
            internal_scratch_in_bytes=1024),
        scratch_types=(
            [pltpu.VMEM((T, _CH), _i32)]
            + [pltpu.VMEM((_CH,), _i32)] * (2 * nr)
            + [pltpu.VMEM((_CH, d), _f32)] * (2 * nr)
            + [pltpu.VMEM((8, _CH), _f32)] * no
            + [pltpu.SemaphoreType.DMA] * (2 * nr + no)
        ),
    )
    def logits_kernel(enc_hbm, pk_hbm, out_hbm, *scr):
        pk_v = scr[0]
        ias = scr[1:1 + nr]
        ibs = scr[1 + nr:1 + 2 * nr]
        abufs = scr[1 + 2 * nr:1 + 3 * nr]
        bbufs = scr[1 + 3 * nr:1 + 4 * nr]
        obufs = scr[1 + 4 * nr:1 + 4 * nr + no]
        sa = scr[1 + 4 * nr + no:1 + 5 * nr + no]
        sb = scr[1 + 5 * nr + no:1 + 6 * nr + no]
        so = scr[1 + 6 * nr + no:]
        c = lax.axis_index("c")
        s = lax.axis_index("s")
        w = s * _NC + c
        iota = _iota16()

        pltpu.sync_copy(pk_hbm.at[pl.ds(w * T, T)], pk_v)

        def unpack(t, k):
            tt = _full16(t)
            ia, ib = ias[k], ibs[k]
            for j in range(ng):
                pk = plsc.load_gather(pk_v, [tt, j * 16 + iota])
                ia[pl.ds(j * 16, 16)] = jnp.bitwise_and(pk, 0xFFFF)
                ib[pl.ds(j * 16, 16)] = lax.shift_right_logical(pk, 16)

        def start_gathers(k):
            pltpu.make_async_copy(enc_hbm.at[ias[k]], abufs[k], sa[k]).start()
            pltpu.make_async_copy(enc_hbm.at[ibs[k]], bbufs[k], sb[k]).start()

        def wait_gathers(k):
            pltpu.make_async_copy(enc_hbm.at[ias[k]], abufs[k], sa[k]).wait()
            pltpu.make_async_copy(enc_hbm.at[ibs[k]], bbufs[k], sb[k]).wait()

        def process(k, q, orow):
            abuf, bbuf, obuf = abufs[k], bbufs[k], obufs[q]

            def col(cix, accs):
                cc = _full16(cix)
                out = []
                for g in range(ng):
                    r = g * 16 + iota
                    va = plsc.load_gather(abuf, [r, cc])
                    vb = plsc.load_gather(bbuf, [r, cc])
                    out.append(accs[g] + va * vb)
                return tuple(out)

            accs = lax.fori_loop(0, d, col,
                                 tuple(jnp.zeros((16,), _f32)
                                       for _ in range(ng)), unroll=4)
            for g in range(ng):
                obuf[orow, pl.ds(g * 16, 16)] = 1.0 / (1.0 + jnp.exp(-accs[g]))

        def flush_out(q, t):
            pltpu.make_async_copy(
                obufs[q], out_hbm.at[pl.ds(w * T + t - 7, 8)], so[q]).start()

        def wait_out(q):
            pltpu.make_async_copy(obufs[q], out_hbm.at[pl.ds(0, 8)],
                                  so[q]).wait()

        for t in range(3):
            unpack(t, t)
            start_gathers(t)

        def body(b, _):
            t0 = b * nb
            for j in range(nb):
                k = j % nr
                q = (j // 8) % no
                tt = t0 + j
                wait_gathers(k)

                @pl.when(tt + 3 < T)
                def _(tt=tt):
                    k2 = (j + 3) % nr
                    unpack(tt + 3, k2)
                    start_gathers(k2)

                if j % 8 == 0:
                    @pl.when(b > 0)
                    def _():
                        wait_out(q)

                process(k, q, j % 8)
                if j % 8 == 7:
                    flush_out(q, tt)
            return 0

        lax.fori_loop(0, T // nb, body, 0)
        for q in range(no):
            wait_out(q)

    return logits_kernel


_BR = 640


def _tc_first(pd_t, xp, w1):
    def body(pd_ref, x_ref, w_ref, dis_ref, hd_ref):
        deg = pd_ref[:, 0:1] + pd_ref[:, 1:2] + 1.0
        dis = jnp.where(deg > 0,
                        lax.rsqrt(jnp.maximum(deg, 1e-12)), 0.0)
        dis_ref[...] = dis
        hd_ref[...] = dis * jnp.dot(x_ref[...], w_ref[...],
                                    preferred_element_type=_f32)

    return pl.pallas_call(
        body,
        grid=(_NPAD // _BR,),
        in_specs=[
            pl.BlockSpec((_BR, 2), lambda i: (i, 0)),
            pl.BlockSpec((_BR, 128), lambda i: (i, 0)),
            pl.BlockSpec((128, 128), lambda i: (0, 0)),
        ],
        out_specs=[
            pl.BlockSpec((_BR, 1), lambda i: (i, 0)),
            pl.BlockSpec((_BR, 128), lambda i: (i, 0)),
        ],
        out_shape=[
            jax.ShapeDtypeStruct((_NPAD, 1), _f32),
            jax.ShapeDtypeStruct((_NPAD, 128), _f32),
        ],
    )(pd_t, xp, w1)


def _tc_combine_mm(pa, pb, hd, dis, b2d, wn):
    dh = pa.shape[2]
    d = hd.shape[1]
    dn = wn.shape[1]

    def body(pa_ref, pb_ref, hd_ref, dis_ref, b_ref, w_ref, out_ref):
        sacc = hd_ref[...] + jnp.concatenate(
            [pa_ref[0] + pa_ref[1], pb_ref[0] + pb_ref[1]], axis=-1)
        xr = jnp.maximum(dis_ref[...] * sacc + b_ref[...], 0.0)
        out_ref[...] = dis_ref[...] * jnp.dot(xr, w_ref[...],
                                              preferred_element_type=_f32)

    return pl.pallas_call(
        body,
        grid=(_NPAD // _BR,),
        in_specs=[
            pl.BlockSpec((2, _BR, dh), lambda i: (0, i, 0)),
            pl.BlockSpec((2, _BR, dh), lambda i: (0, i, 0)),
            pl.BlockSpec((_BR, d), lambda i: (i, 0)),
            pl.BlockSpec((_BR, 1), lambda i: (i, 0)),
            pl.BlockSpec((1, d), lambda i: (0, 0)),
            pl.BlockSpec((d, dn), lambda i: (0, 0)),
        ],
        out_specs=pl.BlockSpec((_BR, dn), lambda i: (i, 0)),
        out_shape=jax.ShapeDtypeStruct((_NPAD, dn), _f32),
    )(pa, pb, hd, dis, b2d, wn)


def _tc_combine_last(p, hd, dis, b2d):
    d = hd.shape[1]

    def body(p_ref, hd_ref, dis_ref, b_ref, out_ref):
        sacc = p_ref[0] + p_ref[1] + hd_ref[...]
        out_ref[...] = dis_ref[...] * sacc + b_ref[...]

    return pl.pallas_call(
        body,
        grid=(_NPAD // _BR,),
        in_specs=[
            pl.BlockSpec((2, _BR, d), lambda i: (0, i, 0)),
            pl.BlockSpec((_BR, d), lambda i: (i, 0)),
            pl.BlockSpec((_BR, 1), lambda i: (i, 0)),
            pl.BlockSpec((1, d), lambda i: (0, 0)),
        ],
        out_specs=pl.BlockSpec((_BR, d), lambda i: (i, 0)),
        out_shape=jax.ShapeDtypeStruct((_NPAD, d), _f32),
    )(p, hd, dis, b2d)


def _pad_chunks(a, t, fill):
    n = _NW * t * _CH
    pad = n - a.shape[0]
    return jnp.concatenate(
        [a, jnp.full((pad,), fill, a.dtype)]).reshape(_NW * t, _CH)


def kernel(x, edge_index, edge_weight, edge_label_index, neg_edge_index,
           W1, b1, W2, b2, W3, b3, W4, b4, W5, b5, W6, b6):
    e = edge_index.shape[1]
    te = -(-(-(-e // _NW)) // _CH)
    te = ((te + 7) // 8) * 8
    l2 = edge_label_index.shape[1] + neg_edge_index.shape[1]
    tl = -(-(-(-l2 // _NW)) // _CH)
    tl = ((tl + 23) // 24) * 24

    src2d = _pad_chunks(edge_index[0], te, 0)
    dst2d = _pad_chunks(edge_index[1], te, 0)
    ww2d = _pad_chunks(edge_weight, te, 0.0)
    em = jnp.stack(
        [src2d, dst2d, lax.bitcast_convert_type(ww2d, _i32)], axis=1)

    eli = jnp.concatenate([edge_label_index, neg_edge_index], axis=1)
    pk2d = _pad_chunks(eli[0] + eli[1] * 65536, tl, 0)

    xp = jnp.concatenate(
        [x, jnp.zeros((_NPAD - x.shape[0], x.shape[1]), _f32)])

    deg_k = _make_deg_kernel(te)
    edge_k = _make_edge_kernel(te, 64)
    logits_k = _make_logits_kernel(tl, 64)

    pd = deg_k(dst2d, ww2d)
    dis, hd = _tc_first(pd.T, xp, W1)

    ws = [W2, W3, W4, W5, W6]
    bs = [b1, b2, b3, b4, b5]
    for i in range(5):
        pa = edge_k(hd[:, :64], em)
        pb = edge_k(hd[:, 64:], em)
        hd = _tc_combine_mm(pa, pb, hd, dis, bs[i].reshape(1, -1), ws[i])

    p6 = edge_k(hd, em)
    enc = _tc_combine_last(p6, hd, dis, b6.reshape(1, -1))

    logits2d = logits_k(enc, pk2d)
    return logits2d.reshape(-1)[:l2]


# speedup vs baseline: 3.9110x; 1.0074x over previous
import functools

import jax
import jax.numpy as jnp
from jax import lax
from jax.experimental import pallas as pl
from jax.experimental.pallas import tpu as pltpu
from jax.experimental.pallas import tpu_sc as plsc

_N = 10000
_NPAD = 10240
_RPT = 640
_NC = 2
_NS = 16
_NW = _NC * _NS
_CH = 128

_i32 = jnp.int32
_f32 = jnp.float32


def _iota16():
    return lax.iota(_i32, 16)


def _full16(v):
    return jnp.full((16,), v, _i32)


def _mesh():
    return plsc.VectorSubcoreMesh(core_axis_name="c", subcore_axis_name="s")


def _make_deg_kernel(T):
    @functools.partial(
        pl.kernel,
        out_type=jax.ShapeDtypeStruct((_NC, _NPAD), _f32),
        mesh=_mesh(),
        compiler_params=pltpu.CompilerParams(
            needs_layout_passes=False, use_tc_tiling_on_sc=False),
        scratch_types=[
            pltpu.VMEM((T, _CH), _i32),
            pltpu.VMEM((T, _CH), _f32),
            pltpu.VMEM((_CH,), _f32),
            pltpu.VMEM_SHARED((_NPAD,), _f32),
        ],
    )
    def deg_kernel(dst_hbm, ww_hbm, out_hbm, dst_v, ww_v, zbuf, deg_sp):
        c = lax.axis_index("c")
        s = lax.axis_index("s")
        w = s * _NC + c
        iota = _iota16()
        z16 = jnp.zeros((16,), _f32)
        for j in range(_CH // 16):
            zbuf[pl.ds(j * 16, 16)] = z16
        for k in range(_RPT // _CH):
            pltpu.sync_copy(zbuf, deg_sp.at[pl.ds(s * _RPT + k * _CH, _CH)])
        plsc.subcore_barrier()

        pltpu.sync_copy(dst_hbm.at[pl.ds(w * T, T)], dst_v)
        pltpu.sync_copy(ww_hbm.at[pl.ds(w * T, T)], ww_v)

        def body(t, _):
            pltpu.sync_copy(ww_v.at[t], deg_sp.at[dst_v.at[t]], add=True)
            return 0

        lax.fori_loop(0, T, body, 0)
        plsc.subcore_barrier()
        pltpu.sync_copy(deg_sp.at[pl.ds(s * _RPT, _RPT)],
                        out_hbm.at[c, pl.ds(s * _RPT, _RPT)])

    return deg_kernel


def _make_edge_kernel(T, d):
    ng = _CH // 16

    nr = 2
    nq = 8

    @functools.partial(
        pl.kernel,
        out_type=jax.ShapeDtypeStruct((_NC, _NPAD, d), _f32),
        mesh=_mesh(),
        compiler_params=pltpu.CompilerParams(
            needs_layout_passes=False, use_tc_tiling_on_sc=False,
            internal_scratch_in_bytes=1024),
        scratch_types=(
            [pltpu.VMEM((3, _CH), _i32)] * nq
            + [pltpu.VMEM((_CH, d), _f32)] * nr
            + [pltpu.VMEM_SHARED((_NPAD, d), _f32)]
            + [pltpu.SemaphoreType.DMA] * (nq + 2 * nr)
        ),
    )
    def edge_kernel(hd_hbm, em_hbm, out_hbm, *scr):
        ms = scr[:nq]
        rows = scr[nq:nq + nr]
        acc_sp = scr[nq + nr]
        msem = scr[nq + nr + 1:nq + nr + 1 + nq]
        gsem = scr[nq + nr + 1 + nq:nq + nr + 1 + nq + nr]
        ssem = scr[nq + nr + 1 + nq + nr:]
        c = lax.axis_index("c")
        s = lax.axis_index("s")
        w = s * _NC + c
        iota = _iota16()
        z16 = jnp.zeros((16,), _f32)

        def zrow(i, _):
            for j in range(d // 16):
                rows[0][i, pl.ds(j * 16, 16)] = z16
            return 0

        lax.fori_loop(0, _CH, zrow, 0)
        for k in range(_RPT // _CH):
            pltpu.sync_copy(rows[0], acc_sp.at[pl.ds(s * _RPT + k * _CH, _CH)])
        plsc.subcore_barrier()

        def start_meta(t, u):
            pltpu.make_async_copy(em_hbm.at[w * T + t], ms[u], msem[u]).start()

        def wait_meta(u):
            pltpu.make_async_copy(em_hbm.at[0], ms[u], msem[u]).wait()

        def start_gather(u, k):
            pltpu.make_async_copy(hd_hbm.at[ms[u].at[0]], rows[k],
                                  gsem[k]).start()

        def wait_gather(u, k):
            pltpu.make_async_copy(hd_hbm.at[ms[u].at[0]], rows[k],
                                  gsem[k]).wait()

        def start_scatter(u, k):
            pltpu.async_copy(rows[k], acc_sp.at[ms[u].at[1]],
                             ssem[k], add=True)

        def wait_scatter(u, k):
            pltpu.make_async_copy(rows[k], acc_sp.at[ms[u].at[1]],
                                  ssem[k]).wait()

        def scale(u, k):
            m, buf = ms[u], rows[k]
            wvs = [plsc.bitcast(
                plsc.load_gather(m, [_full16(2), g * 16 + iota]), _f32)
                   for g in range(ng)]

            @plsc.parallel_loop(0, d, unroll=4)
            def _(cix):
                cc = _full16(cix)
                for g in range(ng):
                    r = g * 16 + iota
                    v = plsc.load_gather(buf, [r, cc])
                    plsc.store_scatter(buf, [r, cc], v * wvs[g])

        for t in range(nq - 2):
            start_meta(t, t)
        wait_meta(0)
        start_gather(0, 0)

        def body(b, _):
            t0 = b * nq
            for j in range(nq):
                u = j
                k = j % nr
                tt = t0 + j
                wait_gather(u, k)

                @pl.when(tt + 1 < T)
                def _(tt=tt):
                    if j == 0:
                        @pl.when(b > 0)
                        def _():
                            wait_scatter((j + 7) % nq, 1 - k)
                    else:
                        wait_scatter(j - 1, 1 - k)
                    wait_meta((j + 1) % nq)
                    start_gather((j + 1) % nq, 1 - k)

                scale(u, k)
                start_scatter(u, k)

                @pl.when(tt + nq - 2 < T)
                def _(tt=tt, u=u):
                    start_meta(tt + nq - 2, (j + nq - 2) % nq)
            return 0

        lax.fori_loop(0, T // nq, body, 0)
        wait_scatter((T - 1) % nq, (T - 1) % nr)
        plsc.subcore_barrier()
        base = s * _RPT
        pltpu.sync_copy(acc_sp.at[pl.ds(base, _RPT)],
                        out_hbm.at[c, pl.ds(base, _RPT)])

    return edge_kernel


def _make_logits_kernel(T, d):
    ng = _CH // 16
    nr = 6
    no = 3
    nb = 24

    @functools.partial(
        pl.kernel,
        out_type=jax.ShapeDtypeStruct((_NW * T, _CH), _f32),
        mesh=_mesh(),
        compiler_params=pltpu.CompilerParams(
            needs_layout_passes=False, use_tc_tiling_on_sc=False,
            internal_scratch_in_bytes=1024),
        scratch_types=(
            [pltpu.VMEM((T, _CH), _i32)]
            + [pltpu.VMEM((_CH,), _i32)] * (2 * nr)
            + [pltpu.VMEM((_CH, d), _f32)] * (2 * nr)
            + [pltpu.VMEM((8, _CH), _f32)] * no
            + [pltpu.SemaphoreType.DMA] * (2 * nr + no)
        ),
    )
    def logits_kernel(enc_hbm, pk_hbm, out_hbm, *scr):
        pk_v = scr[0]
        ias = scr[1:1 + nr]
        ibs = scr[1 + nr:1 + 2 * nr]
        abufs = scr[1 + 2 * nr:1 + 3 * nr]
        bbufs = scr[1 + 3 * nr:1 + 4 * nr]
        obufs = scr[1 + 4 * nr:1 + 4 * nr + no]
        sa = scr[1 + 4 * nr + no:1 + 5 * nr + no]
        sb = scr[1 + 5 * nr + no:1 + 6 * nr + no]
        so = scr[1 + 6 * nr + no:]
        c = lax.axis_index("c")
        s = lax.axis_index("s")
        w = s * _NC + c
        iota = _iota16()

        pltpu.sync_copy(pk_hbm.at[pl.ds(w * T, T)], pk_v)

        def unpack(t, k):
            tt = _full16(t)
            ia, ib = ias[k], ibs[k]
            for j in range(ng):
                pk = plsc.load_gather(pk_v, [tt, j * 16 + iota])
                ia[pl.ds(j * 16, 16)] = jnp.bitwise_and(pk, 0xFFFF)
                ib[pl.ds(j * 16, 16)] = lax.shift_right_logical(pk, 16)

        def start_gathers(k):
            pltpu.make_async_copy(enc_hbm.at[ias[k]], abufs[k], sa[k]).start()
            pltpu.make_async_copy(enc_hbm.at[ibs[k]], bbufs[k], sb[k]).start()

        def wait_gathers(k):
            pltpu.make_async_copy(enc_hbm.at[ias[k]], abufs[k], sa[k]).wait()
            pltpu.make_async_copy(enc_hbm.at[ibs[k]], bbufs[k], sb[k]).wait()

        def process(k, q, orow):
            abuf, bbuf, obuf = abufs[k], bbufs[k], obufs[q]

            def col(cix, accs):
                cc = _full16(cix)
                out = []
                for g in range(ng):
                    r = g * 16 + iota
                    va = plsc.load_gather(abuf, [r, cc])
                    vb = plsc.load_gather(bbuf, [r, cc])
                    out.append(accs[g] + va * vb)
                return tuple(out)

            accs = lax.fori_loop(0, d, col,
                                 tuple(jnp.zeros((16,), _f32)
                                       for _ in range(ng)), unroll=4)
            for g in range(ng):
                obuf[orow, pl.ds(g * 16, 16)] = 1.0 / (1.0 + jnp.exp(-accs[g]))

        def flush_out(q, t):
            pltpu.make_async_copy(
                obufs[q], out_hbm.at[pl.ds(w * T + t - 7, 8)], so[q]).start()

        def wait_out(q):
            pltpu.make_async_copy(obufs[q], out_hbm.at[pl.ds(0, 8)],
                                  so[q]).wait()

        for t in range(3):
            unpack(t, t)
            start_gathers(t)

        def body(b, _):
            t0 = b * nb
            for j in range(nb):
                k = j % nr
                q = (j // 8) % no
                tt = t0 + j
                wait_gathers(k)

                @pl.when(tt + 3 < T)
                def _(tt=tt):
                    k2 = (j + 3) % nr
                    unpack(tt + 3, k2)
                    start_gathers(k2)

                if j % 8 == 0:
                    @pl.when(b > 0)
                    def _():
                        wait_out(q)

                process(k, q, j % 8)
                if j % 8 == 7:
                    flush_out(q, tt)
            return 0

        lax.fori_loop(0, T // nb, body, 0)
        for q in range(no):
            wait_out(q)

    return logits_kernel


_BR = 640


def _tc_first(pd_t, xp, w1):
    def body(pd_ref, x_ref, w_ref, dis_ref, hd_ref):
        deg = pd_ref[:, 0:1] + pd_ref[:, 1:2] + 1.0
        dis = jnp.where(deg > 0,
                        lax.rsqrt(jnp.maximum(deg, 1e-12)), 0.0)
        dis_ref[...] = dis
        hd_ref[...] = dis * jnp.dot(x_ref[...], w_ref[...],
                                    preferred_element_type=_f32)

    return pl.pallas_call(
        body,
        grid=(_NPAD // _BR,),
        in_specs=[
            pl.BlockSpec((_BR, 2), lambda i: (i, 0)),
            pl.BlockSpec((_BR, 128), lambda i: (i, 0)),
            pl.BlockSpec((128, 128), lambda i: (0, 0)),
        ],
        out_specs=[
            pl.BlockSpec((_BR, 1), lambda i: (i, 0)),
            pl.BlockSpec((_BR, 128), lambda i: (i, 0)),
        ],
        out_shape=[
            jax.ShapeDtypeStruct((_NPAD, 1), _f32),
            jax.ShapeDtypeStruct((_NPAD, 128), _f32),
        ],
    )(pd_t, xp, w1)


def _tc_combine_mm(p, hd, dis, b2d, wn):
    d = hd.shape[1]
    dn = wn.shape[1]

    def body(p_ref, hd_ref, dis_ref, b_ref, w_ref, out_ref):
        sacc = hd_ref[...] + p_ref[0] + p_ref[1]
        xr = jnp.maximum(dis_ref[...] * sacc + b_ref[...], 0.0)
        out_ref[...] = dis_ref[...] * jnp.dot(xr, w_ref[...],
                                              preferred_element_type=_f32)

    return pl.pallas_call(
        body,
        grid=(_NPAD // _BR,),
        in_specs=[
            pl.BlockSpec((2, _BR, d), lambda i: (0, i, 0)),
            pl.BlockSpec((_BR, d), lambda i: (i, 0)),
            pl.BlockSpec((_BR, 1), lambda i: (i, 0)),
            pl.BlockSpec((1, d), lambda i: (0, 0)),
            pl.BlockSpec((d, dn), lambda i: (0, 0)),
        ],
        out_specs=pl.BlockSpec((_BR, dn), lambda i: (i, 0)),
        out_shape=jax.ShapeDtypeStruct((_NPAD, dn), _f32),
    )(p, hd, dis, b2d, wn)


def _tc_combine_last(p, hd, dis, b2d):
    d = hd.shape[1]

    def body(p_ref, hd_ref, dis_ref, b_ref, out_ref):
        sacc = p_ref[0] + p_ref[1] + hd_ref[...]
        out_ref[...] = dis_ref[...] * sacc + b_ref[...]

    return pl.pallas_call(
        body,
        grid=(_NPAD // _BR,),
        in_specs=[
            pl.BlockSpec((2, _BR, d), lambda i: (0, i, 0)),
            pl.BlockSpec((_BR, d), lambda i: (i, 0)),
            pl.BlockSpec((_BR, 1), lambda i: (i, 0)),
            pl.BlockSpec((1, d), lambda i: (0, 0)),
        ],
        out_specs=pl.BlockSpec((_BR, d), lambda i: (i, 0)),
        out_shape=jax.ShapeDtypeStruct((_NPAD, d), _f32),
    )(p, hd, dis, b2d)


def _pad_chunks(a, t, fill):
    n = _NW * t * _CH
    pad = n - a.shape[0]
    return jnp.concatenate(
        [a, jnp.full((pad,), fill, a.dtype)]).reshape(_NW * t, _CH)


def kernel(x, edge_index, edge_weight, edge_label_index, neg_edge_index,
           W1, b1, W2, b2, W3, b3, W4, b4, W5, b5, W6, b6):
    e = edge_index.shape[1]
    te = -(-(-(-e // _NW)) // _CH)
    te = ((te + 7) // 8) * 8
    l2 = edge_label_index.shape[1] + neg_edge_index.shape[1]
    tl = -(-(-(-l2 // _NW)) // _CH)
    tl = ((tl + 23) // 24) * 24

    src2d = _pad_chunks(edge_index[0], te, 0)
    dst2d = _pad_chunks(edge_index[1], te, 0)
    ww2d = _pad_chunks(edge_weight, te, 0.0)
    em = jnp.stack(
        [src2d, dst2d, lax.bitcast_convert_type(ww2d, _i32)], axis=1)

    eli = jnp.concatenate([edge_label_index, neg_edge_index], axis=1)
    pk2d = _pad_chunks(eli[0] + eli[1] * 65536, tl, 0)

    xp = jnp.concatenate(
        [x, jnp.zeros((_NPAD - x.shape[0], x.shape[1]), _f32)])

    deg_k = _make_deg_kernel(te)
    edge_k128 = _make_edge_kernel(te, 128)
    edge_k64 = _make_edge_kernel(te, 64)
    logits_k = _make_logits_kernel(tl, 64)

    pd = deg_k(dst2d, ww2d)
    dis, hd = _tc_first(pd.T, xp, W1)

    ws = [W2, W3, W4, W5, W6]
    bs = [b1, b2, b3, b4, b5]
    for i in range(5):
        p = edge_k128(hd, em)
        hd = _tc_combine_mm(p, hd, dis, bs[i].reshape(1, -1), ws[i])

    p6 = edge_k64(hd, em)
    enc = _tc_combine_last(p6, hd, dis, b6.reshape(1, -1))

    logits2d = logits_k(enc, pk2d)
    return logits2d.reshape(-1)[:l2]
